# Initial kernel scaffold; baseline (speedup 1.0000x reference)
import jax
import jax.numpy as jnp
from jax.experimental import pallas as pl

D_MODEL = 2048
N_HEADS = 16
D_K = 128
SEQ = 2048
HI = 4
DI = 64
K_SEL = 512

_QI = HI * DI
_N_COLS = 3 * D_MODEL + _QI + DI + HI
_N_PAD = 6656
_BM = 256
_BN = 512


def _p1_proj_rope(x_ref, w_ref, cos_ref, sin_ref, rot_ref, o_ref):
    acc = jnp.dot(x_ref[...], w_ref[...], preferred_element_type=jnp.float32)
    n = pl.program_id(1)

    @pl.when(n < 8)
    def _():
        rot = jnp.dot(acc, rot_ref[...], preferred_element_type=jnp.float32)
        o_ref[...] = acc * cos_ref[...] + rot * sin_ref[...]

    @pl.when(n >= 8)
    def _():
        o_ref[...] = acc


def _p2_index_topk(qw_ref, all_ref, o_ref):
    qw = qw_ref[...]
    ki = all_ref[:, _QI:_QI + DI]
    acc = None
    for h in range(HI):
        qh = qw[:, h * DI:(h + 1) * DI]
        d = jax.lax.dot_general(qh, ki, (((1,), (1,)), ((), ())),
                                preferred_element_type=jnp.float32)
        a = jnp.maximum(d, 0.0) * qw[:, _QI + DI + h:_QI + DI + h + 1]
        acc = a if acc is None else acc + a

    m = pl.program_id(0)
    row = m * _BM + jax.lax.broadcasted_iota(jnp.int32, (_BM, SEQ), 0)
    col = jax.lax.broadcasted_iota(jnp.int32, (_BM, SEQ), 1)
    scores = jnp.where(col > row, jnp.float32(-1e9), acc)

    z = jnp.where(scores == 0.0, jnp.float32(0.0), scores)
    bits = jax.lax.bitcast_convert_type(z, jnp.int32)
    keys = jnp.where(bits < 0, bits ^ jnp.int32(0x7FFFFFFF), bits)

    lo = jnp.min(keys, axis=1, keepdims=True)
    hi = jnp.max(keys, axis=1, keepdims=True)

    def body(_, lh):
        lo, hi = lh
        x = lo ^ hi
        mid = (lo & hi) + (x >> 1) + (x & 1)
        cnt = jnp.sum((keys >= mid).astype(jnp.int32), axis=1, keepdims=True)
        take = cnt >= K_SEL
        return jnp.where(take, mid, lo), jnp.where(take, hi, mid - 1)

    tau, _ = jax.lax.fori_loop(0, 32, body, (lo, hi))

    gt = keys > tau
    cnt_gt = jnp.sum(gt.astype(jnp.int32), axis=1, keepdims=True)
    ties = keys == tau
    tcum = jnp.cumsum(ties.astype(jnp.int32), axis=1)
    sel = gt | (ties & (tcum <= (K_SEL - cnt_gt)))
    o_ref[...] = jnp.where(sel, jnp.float32(0.0), jnp.float32(-1e30))


def _p3_sdpa(q_ref, k_ref, v_ref, mask_ref, o_ref):
    s = jax.lax.dot_general(q_ref[...], k_ref[...], (((1,), (1,)), ((), ())),
                            preferred_element_type=jnp.float32)
    s = s / jnp.sqrt(jnp.float32(D_K)) + mask_ref[...]
    mx = jnp.max(s, axis=1, keepdims=True)
    p = jnp.exp(s - mx)
    l = jnp.sum(p, axis=1, keepdims=True)
    o = jnp.dot(p, v_ref[...], preferred_element_type=jnp.float32) / l
    o_ref[...] = o


def _p4_matmul(a_ref, w_ref, o_ref):
    o_ref[...] = jnp.dot(a_ref[...], w_ref[...],
                         preferred_element_type=jnp.float32)


def kernel(x, Wqkv, Wo, Wq_idx, Wk_idx, Ww_idx):
    b, s, _ = x.shape
    x2 = x[0]

    WT = jnp.concatenate([Wqkv, Wq_idx, Wk_idx, Ww_idx], axis=0)
    WT = jnp.pad(WT, ((0, _N_PAD - _N_COLS), (0, 0))).T

    theta = 1.0 / (10000.0 ** (jnp.arange(0, D_K, 2, dtype=jnp.float32) / D_K))
    pos = jnp.arange(SEQ, dtype=jnp.float32)
    freqs = pos[:, None] * theta[None, :]
    cos2 = jnp.repeat(jnp.cos(freqs), 2, axis=1)
    sin2 = jnp.repeat(jnp.sin(freqs), 2, axis=1)
    cos512 = jnp.tile(cos2, (1, _BN // D_K))
    sin512 = jnp.tile(sin2, (1, _BN // D_K))
    r128 = (jnp.zeros((D_K, D_K), jnp.float32)
            .at[2 * jnp.arange(64) + 1, 2 * jnp.arange(64)].set(-1.0)
            .at[2 * jnp.arange(64), 2 * jnp.arange(64) + 1].set(1.0))
    r512 = jax.scipy.linalg.block_diag(*([r128] * (_BN // D_K)))

    proj = pl.pallas_call(
        _p1_proj_rope,
        grid=(SEQ // _BM, _N_PAD // _BN),
        in_specs=[
            pl.BlockSpec((_BM, D_MODEL), lambda m, n: (m, 0)),
            pl.BlockSpec((D_MODEL, _BN), lambda m, n: (0, n)),
            pl.BlockSpec((_BM, _BN), lambda m, n: (m, 0)),
            pl.BlockSpec((_BM, _BN), lambda m, n: (m, 0)),
            pl.BlockSpec((_BN, _BN), lambda m, n: (0, 0)),
        ],
        out_specs=pl.BlockSpec((_BM, _BN), lambda m, n: (m, n)),
        out_shape=jax.ShapeDtypeStruct((SEQ, _N_PAD), jnp.float32),
    )(x2, WT, cos512, sin512, r512)

    maskadd = pl.pallas_call(
        _p2_index_topk,
        grid=(SEQ // _BM,),
        in_specs=[
            pl.BlockSpec((_BM, _BN), lambda m: (m, 12)),
            pl.BlockSpec((SEQ, _BN), lambda m: (0, 12)),
        ],
        out_specs=pl.BlockSpec((_BM, SEQ), lambda m: (m, 0)),
        out_shape=jax.ShapeDtypeStruct((SEQ, SEQ), jnp.float32),
    )(proj, proj)

    attn = pl.pallas_call(
        _p3_sdpa,
        grid=(N_HEADS, SEQ // _BM),
        in_specs=[
            pl.BlockSpec((_BM, D_K), lambda h, m: (m, h)),
            pl.BlockSpec((SEQ, D_K), lambda h, m: (0, 16 + h)),
            pl.BlockSpec((SEQ, D_K), lambda h, m: (0, 32 + h)),
            pl.BlockSpec((_BM, SEQ), lambda h, m: (m, 0)),
        ],
        out_specs=pl.BlockSpec((_BM, D_K), lambda h, m: (m, h)),
        out_shape=jax.ShapeDtypeStruct((SEQ, D_MODEL), jnp.float32),
    )(proj, proj, proj, maskadd)

    out = pl.pallas_call(
        _p4_matmul,
        grid=(SEQ // _BM, D_MODEL // _BN),
        in_specs=[
            pl.BlockSpec((_BM, D_MODEL), lambda m, n: (m, 0)),
            pl.BlockSpec((D_MODEL, _BN), lambda m, n: (0, n)),
        ],
        out_specs=pl.BlockSpec((_BM, _BN), lambda m, n: (m, n)),
        out_shape=jax.ShapeDtypeStruct((SEQ, D_MODEL), jnp.float32),
    )(attn, Wo.T)

    return out.reshape(b, s, D_MODEL)


# speedup vs baseline: 7.8685x; 7.8685x over previous
import jax
import jax.numpy as jnp
from jax.experimental import pallas as pl

D_MODEL = 2048
N_HEADS = 16
D_K = 128
SEQ = 2048
HI = 4
DI = 64
K_SEL = 512

_QI = HI * DI
_N_COLS = 3 * D_MODEL + _QI + DI + HI
_N_PAD = 6656
_BM = 256
_BN = 512


def _p1_proj_rope(x_ref, w_ref, cos_ref, sin_ref, rot_ref, o_ref):
    acc = jnp.dot(x_ref[...], w_ref[...], preferred_element_type=jnp.float32)
    n = pl.program_id(1)

    @pl.when(n < 8)
    def _():
        rot = jnp.dot(acc, rot_ref[...], preferred_element_type=jnp.float32)
        o_ref[...] = acc * cos_ref[...] + rot * sin_ref[...]

    @pl.when(n >= 8)
    def _():
        o_ref[...] = acc


def _p2_index_topk(qw_ref, all_ref, o_ref):
    qw = qw_ref[...]
    ki = all_ref[:, _QI:_QI + DI]
    acc = None
    for h in range(HI):
        qh = qw[:, h * DI:(h + 1) * DI]
        d = jax.lax.dot_general(qh, ki, (((1,), (1,)), ((), ())),
                                preferred_element_type=jnp.float32)
        a = jnp.maximum(d, 0.0) * qw[:, _QI + DI + h:_QI + DI + h + 1]
        acc = a if acc is None else acc + a

    m = pl.program_id(0)
    row = m * _BM + jax.lax.broadcasted_iota(jnp.int32, (_BM, SEQ), 0)
    col = jax.lax.broadcasted_iota(jnp.int32, (_BM, SEQ), 1)
    scores = jnp.where(col > row, jnp.float32(-1e9), acc)

    z = jnp.where(scores == 0.0, jnp.float32(0.0), scores)
    bits = jax.lax.bitcast_convert_type(z, jnp.int32)
    keys = jnp.where(bits < 0, bits ^ jnp.int32(0x7FFFFFFF), bits)

    lo = jnp.min(keys, axis=1, keepdims=True)
    hi = jnp.max(keys, axis=1, keepdims=True)

    def body(_, lh):
        lo, hi = lh
        x = lo ^ hi
        mid = (lo & hi) + (x >> 1) + (x & 1)
        cnt = jnp.sum((keys >= mid).astype(jnp.int32), axis=1, keepdims=True)
        take = cnt >= K_SEL
        return jnp.where(take, mid, lo), jnp.where(take, hi, mid - 1)

    tau, _ = jax.lax.fori_loop(0, 32, body, (lo, hi))

    gt = keys > tau
    cnt_gt = jnp.sum(gt.astype(jnp.int32), axis=1, keepdims=True)
    ties = keys == tau
    need = K_SEL - cnt_gt

    def body2(_, lh):
        lo2, hi2 = lh
        mid = (lo2 + hi2) >> 1
        cnt = jnp.sum((ties & (col <= mid)).astype(jnp.int32),
                      axis=1, keepdims=True)
        ok = cnt >= need
        return jnp.where(ok, lo2, mid + 1), jnp.where(ok, mid, hi2)

    c_sel, _ = jax.lax.fori_loop(
        0, 11, body2,
        (jnp.zeros_like(cnt_gt), jnp.full_like(cnt_gt, SEQ - 1)))
    sel = gt | (ties & (col <= c_sel))
    o_ref[...] = jnp.where(sel, jnp.float32(0.0), jnp.float32(-1e30))


def _p3_sdpa(q_ref, k_ref, v_ref, mask_ref, o_ref):
    s = jax.lax.dot_general(q_ref[...], k_ref[...], (((1,), (1,)), ((), ())),
                            preferred_element_type=jnp.float32)
    s = s / jnp.sqrt(jnp.float32(D_K)) + mask_ref[...]
    mx = jnp.max(s, axis=1, keepdims=True)
    p = jnp.exp(s - mx)
    l = jnp.sum(p, axis=1, keepdims=True)
    o = jnp.dot(p, v_ref[...], preferred_element_type=jnp.float32) / l
    o_ref[...] = o


def _p4_matmul(a_ref, w_ref, o_ref):
    o_ref[...] = jnp.dot(a_ref[...], w_ref[...],
                         preferred_element_type=jnp.float32)


def kernel(x, Wqkv, Wo, Wq_idx, Wk_idx, Ww_idx):
    b, s, _ = x.shape
    x2 = x[0]

    WT = jnp.concatenate([Wqkv, Wq_idx, Wk_idx, Ww_idx], axis=0)
    WT = jnp.pad(WT, ((0, _N_PAD - _N_COLS), (0, 0))).T

    theta = 1.0 / (10000.0 ** (jnp.arange(0, D_K, 2, dtype=jnp.float32) / D_K))
    pos = jnp.arange(SEQ, dtype=jnp.float32)
    freqs = pos[:, None] * theta[None, :]
    cos2 = jnp.repeat(jnp.cos(freqs), 2, axis=1)
    sin2 = jnp.repeat(jnp.sin(freqs), 2, axis=1)
    cos512 = jnp.tile(cos2, (1, _BN // D_K))
    sin512 = jnp.tile(sin2, (1, _BN // D_K))
    r128 = (jnp.zeros((D_K, D_K), jnp.float32)
            .at[2 * jnp.arange(64) + 1, 2 * jnp.arange(64)].set(-1.0)
            .at[2 * jnp.arange(64), 2 * jnp.arange(64) + 1].set(1.0))
    r512 = jax.scipy.linalg.block_diag(*([r128] * (_BN // D_K)))

    proj = pl.pallas_call(
        _p1_proj_rope,
        grid=(SEQ // _BM, _N_PAD // _BN),
        in_specs=[
            pl.BlockSpec((_BM, D_MODEL), lambda m, n: (m, 0)),
            pl.BlockSpec((D_MODEL, _BN), lambda m, n: (0, n)),
            pl.BlockSpec((_BM, _BN), lambda m, n: (m, 0)),
            pl.BlockSpec((_BM, _BN), lambda m, n: (m, 0)),
            pl.BlockSpec((_BN, _BN), lambda m, n: (0, 0)),
        ],
        out_specs=pl.BlockSpec((_BM, _BN), lambda m, n: (m, n)),
        out_shape=jax.ShapeDtypeStruct((SEQ, _N_PAD), jnp.float32),
    )(x2, WT, cos512, sin512, r512)

    maskadd = pl.pallas_call(
        _p2_index_topk,
        grid=(SEQ // _BM,),
        in_specs=[
            pl.BlockSpec((_BM, _BN), lambda m: (m, 12)),
            pl.BlockSpec((SEQ, _BN), lambda m: (0, 12)),
        ],
        out_specs=pl.BlockSpec((_BM, SEQ), lambda m: (m, 0)),
        out_shape=jax.ShapeDtypeStruct((SEQ, SEQ), jnp.float32),
    )(proj, proj)

    attn = pl.pallas_call(
        _p3_sdpa,
        grid=(N_HEADS, SEQ // _BM),
        in_specs=[
            pl.BlockSpec((_BM, D_K), lambda h, m: (m, h)),
            pl.BlockSpec((SEQ, D_K), lambda h, m: (0, 16 + h)),
            pl.BlockSpec((SEQ, D_K), lambda h, m: (0, 32 + h)),
            pl.BlockSpec((_BM, SEQ), lambda h, m: (m, 0)),
        ],
        out_specs=pl.BlockSpec((_BM, D_K), lambda h, m: (m, h)),
        out_shape=jax.ShapeDtypeStruct((SEQ, D_MODEL), jnp.float32),
    )(proj, proj, proj, maskadd)

    out = pl.pallas_call(
        _p4_matmul,
        grid=(SEQ // _BM, D_MODEL // _BN),
        in_specs=[
            pl.BlockSpec((_BM, D_MODEL), lambda m, n: (m, 0)),
            pl.BlockSpec((D_MODEL, _BN), lambda m, n: (0, n)),
        ],
        out_specs=pl.BlockSpec((_BM, _BN), lambda m, n: (m, n)),
        out_shape=jax.ShapeDtypeStruct((SEQ, D_MODEL), jnp.float32),
    )(attn, Wo.T)

    return out.reshape(b, s, D_MODEL)


# speedup vs baseline: 12.0172x; 1.5273x over previous
import jax
import jax.numpy as jnp
from jax.experimental import pallas as pl

D_MODEL = 2048
N_HEADS = 16
D_K = 128
SEQ = 2048
HI = 4
DI = 64
K_SEL = 512

_QI = HI * DI
_N_COLS = 3 * D_MODEL + _QI + DI + HI
_N_PAD = 6656
_BM = 256
_BN = 512
_HG = 8


def _p1_proj_rope(x_ref, w_ref, cos_ref, sin_ref, rot_ref, o_ref):
    acc = jnp.dot(x_ref[...], w_ref[...], preferred_element_type=jnp.float32)
    n = pl.program_id(0)

    @pl.when(n < 8)
    def _():
        rot = jnp.dot(acc, rot_ref[...], preferred_element_type=jnp.float32)
        o_ref[...] = acc * cos_ref[...] + rot * sin_ref[...]

    @pl.when(n >= 8)
    def _():
        o_ref[...] = acc


def _p2_index_topk(qw_ref, ki_ref, o_ref):
    m = pl.program_id(0)
    col = jax.lax.broadcasted_iota(jnp.int32, (_BM, SEQ), 1)

    @pl.when(m < 2)
    def _():
        o_ref[...] = (col < K_SEL).astype(jnp.int8)

    @pl.when(m >= 2)
    def _():
        qw = qw_ref[...]
        ki = ki_ref[:, :DI]
        acc = None
        for h in range(HI):
            qh = qw[:, h * DI:(h + 1) * DI]
            d = jax.lax.dot_general(qh, ki, (((1,), (1,)), ((), ())),
                                    preferred_element_type=jnp.float32)
            a = jnp.maximum(d, 0.0) * qw[:, _QI + DI + h:_QI + DI + h + 1]
            acc = a if acc is None else acc + a

        row = m * _BM + jax.lax.broadcasted_iota(jnp.int32, (_BM, SEQ), 0)
        scores = jnp.where(col > row, jnp.float32(-1e9), acc)

        z = jnp.where(scores == 0.0, jnp.float32(0.0), scores)
        bits = jax.lax.bitcast_convert_type(z, jnp.int32)
        keys = jnp.where(bits < 0, bits ^ jnp.int32(0x7FFFFFFF), bits)

        lo = jnp.min(keys, axis=1, keepdims=True)
        hi = jnp.max(keys, axis=1, keepdims=True)

        def body(_, lh):
            lo, hi = lh
            x = lo ^ hi
            mid = (lo & hi) + (x >> 1) + (x & 1)
            cnt = jnp.sum((keys >= mid).astype(jnp.int32),
                          axis=1, keepdims=True)
            take = cnt >= K_SEL
            return jnp.where(take, mid, lo), jnp.where(take, hi, mid - 1)

        tau, _ = jax.lax.fori_loop(0, 32, body, (lo, hi))

        gt = keys > tau
        cnt_gt = jnp.sum(gt.astype(jnp.int32), axis=1, keepdims=True)
        ties = keys == tau
        need = K_SEL - cnt_gt

        def body2(_, lh):
            lo2, hi2 = lh
            mid = (lo2 + hi2) >> 1
            cnt = jnp.sum((ties & (col <= mid)).astype(jnp.int32),
                          axis=1, keepdims=True)
            ok = cnt >= need
            return jnp.where(ok, lo2, mid + 1), jnp.where(ok, mid, hi2)

        c_sel, _ = jax.lax.fori_loop(
            0, 11, body2,
            (jnp.zeros_like(cnt_gt), jnp.full_like(cnt_gt, SEQ - 1)))
        sel = gt | (ties & (col <= c_sel))
        o_ref[...] = sel.astype(jnp.int8)


def _p3_sdpa(q_ref, k_ref, v_ref, mask_ref, o_ref):
    keep = mask_ref[...] != 0
    for h in range(_HG):
        sl = slice(h * D_K, (h + 1) * D_K)
        s = jax.lax.dot_general(q_ref[:, sl], k_ref[:, sl],
                                (((1,), (1,)), ((), ())),
                                preferred_element_type=jnp.float32)
        s = jnp.where(keep, s / jnp.sqrt(jnp.float32(D_K)), jnp.float32(-1e30))
        mx = jnp.max(s, axis=1, keepdims=True)
        p = jnp.exp(s - mx)
        l = jnp.sum(p, axis=1, keepdims=True)
        o_ref[:, sl] = jnp.dot(p, v_ref[:, sl],
                               preferred_element_type=jnp.float32) / l


def _p4_matmul(a_ref, w_ref, o_ref):
    o_ref[...] = jnp.dot(a_ref[...], w_ref[...],
                         preferred_element_type=jnp.float32)


def kernel(x, Wqkv, Wo, Wq_idx, Wk_idx, Ww_idx):
    b, s, _ = x.shape
    x2 = x[0]

    WT = jnp.concatenate([Wqkv, Wq_idx, Wk_idx, Ww_idx], axis=0)
    WT = jnp.pad(WT, ((0, _N_PAD - _N_COLS), (0, 0))).T

    theta = 1.0 / (10000.0 ** (jnp.arange(0, D_K, 2, dtype=jnp.float32) / D_K))
    pos = jnp.arange(SEQ, dtype=jnp.float32)
    freqs = pos[:, None] * theta[None, :]
    cos2 = jnp.repeat(jnp.cos(freqs), 2, axis=1)
    sin2 = jnp.repeat(jnp.sin(freqs), 2, axis=1)
    cos512 = jnp.tile(cos2, (1, _BN // D_K))
    sin512 = jnp.tile(sin2, (1, _BN // D_K))
    r128 = (jnp.zeros((D_K, D_K), jnp.float32)
            .at[2 * jnp.arange(64) + 1, 2 * jnp.arange(64)].set(-1.0)
            .at[2 * jnp.arange(64), 2 * jnp.arange(64) + 1].set(1.0))
    r512 = jax.scipy.linalg.block_diag(*([r128] * (_BN // D_K)))

    proj = pl.pallas_call(
        _p1_proj_rope,
        grid=(_N_PAD // _BN,),
        in_specs=[
            pl.BlockSpec((SEQ, D_MODEL), lambda n: (0, 0)),
            pl.BlockSpec((D_MODEL, _BN), lambda n: (0, n)),
            pl.BlockSpec((SEQ, _BN), lambda n: (0, 0)),
            pl.BlockSpec((SEQ, _BN), lambda n: (0, 0)),
            pl.BlockSpec((_BN, _BN), lambda n: (0, 0)),
        ],
        out_specs=pl.BlockSpec((SEQ, _BN), lambda n: (0, n)),
        out_shape=jax.ShapeDtypeStruct((SEQ, _N_PAD), jnp.float32),
    )(x2, WT, cos512, sin512, r512)

    selmask = pl.pallas_call(
        _p2_index_topk,
        grid=(SEQ // _BM,),
        in_specs=[
            pl.BlockSpec((_BM, _BN), lambda m: (m, 12)),
            pl.BlockSpec((SEQ, D_K), lambda m: (0, 50)),
        ],
        out_specs=pl.BlockSpec((_BM, SEQ), lambda m: (m, 0)),
        out_shape=jax.ShapeDtypeStruct((SEQ, SEQ), jnp.int8),
    )(proj, proj)

    hgw = _HG * D_K
    attn = pl.pallas_call(
        _p3_sdpa,
        grid=(N_HEADS // _HG, SEQ // _BM),
        in_specs=[
            pl.BlockSpec((_BM, hgw), lambda g, m: (m, g)),
            pl.BlockSpec((SEQ, hgw), lambda g, m: (0, 2 + g)),
            pl.BlockSpec((SEQ, hgw), lambda g, m: (0, 4 + g)),
            pl.BlockSpec((_BM, SEQ), lambda g, m: (m, 0)),
        ],
        out_specs=pl.BlockSpec((_BM, hgw), lambda g, m: (m, g)),
        out_shape=jax.ShapeDtypeStruct((SEQ, D_MODEL), jnp.float32),
    )(proj, proj, proj, selmask)

    out = pl.pallas_call(
        _p4_matmul,
        grid=(D_MODEL // _BN,),
        in_specs=[
            pl.BlockSpec((SEQ, D_MODEL), lambda n: (0, 0)),
            pl.BlockSpec((D_MODEL, _BN), lambda n: (0, n)),
        ],
        out_specs=pl.BlockSpec((SEQ, _BN), lambda n: (0, n)),
        out_shape=jax.ShapeDtypeStruct((SEQ, D_MODEL), jnp.float32),
    )(attn, Wo.T)

    return out.reshape(b, s, D_MODEL)


# speedup vs baseline: 13.0112x; 1.0827x over previous
import functools

import jax
import jax.numpy as jnp
from jax.experimental import pallas as pl

D_MODEL = 2048
N_HEADS = 16
D_K = 128
SEQ = 2048
HI = 4
DI = 64
K_SEL = 512

_QI = HI * DI
_N_COLS = 3 * D_MODEL + _QI + DI + HI
_N_PAD = 6656
_BM = 256
_BN = 512
_HG = 8
_HGW = _HG * D_K


def _p1_proj_rope(x_ref, w_ref, cos_ref, sin_ref, rot_ref, o_ref):
    acc = jnp.dot(x_ref[...], w_ref[...], preferred_element_type=jnp.float32)
    n = pl.program_id(0)

    @pl.when(n < 8)
    def _():
        rot = jnp.dot(acc, rot_ref[...], preferred_element_type=jnp.float32)
        o_ref[...] = acc * cos_ref[...] + rot * sin_ref[...]

    @pl.when(n >= 8)
    def _():
        o_ref[...] = acc


def _p2_index_topk(qw_ref, ki_ref, o_ref, *, row0, width, col_iters):
    qw = qw_ref[...]
    ki = ki_ref[:, :DI]
    acc = None
    for h in range(HI):
        qh = qw[:, h * DI:(h + 1) * DI]
        d = jax.lax.dot_general(qh, ki, (((1,), (1,)), ((), ())),
                                preferred_element_type=jnp.float32)
        a = jnp.maximum(d, 0.0) * qw[:, _QI + DI + h:_QI + DI + h + 1]
        acc = a if acc is None else acc + a

    m = pl.program_id(0)
    row = row0 + m * _BM + jax.lax.broadcasted_iota(jnp.int32, (_BM, width), 0)
    col = jax.lax.broadcasted_iota(jnp.int32, (_BM, width), 1)
    scores = jnp.where(col > row, jnp.float32(-1e9), acc)

    z = jnp.where(scores == 0.0, jnp.float32(0.0), scores)
    bits = jax.lax.bitcast_convert_type(z, jnp.int32)
    keys = jnp.where(bits < 0, bits ^ jnp.int32(0x7FFFFFFF), bits)

    lo = jnp.min(keys, axis=1, keepdims=True)
    hi = jnp.max(keys, axis=1, keepdims=True)

    def body(_, lh):
        lo, hi = lh
        x = lo ^ hi
        mid = (lo & hi) + (x >> 1) + (x & 1)
        cnt = jnp.sum((keys >= mid).astype(jnp.int32), axis=1, keepdims=True)
        take = cnt >= K_SEL
        return jnp.where(take, mid, lo), jnp.where(take, hi, mid - 1)

    tau, _ = jax.lax.fori_loop(0, 32, body, (lo, hi))

    gt = keys > tau
    cnt_gt = jnp.sum(gt.astype(jnp.int32), axis=1, keepdims=True)
    ties = keys == tau
    need = K_SEL - cnt_gt

    def body2(_, lh):
        lo2, hi2 = lh
        mid = (lo2 + hi2) >> 1
        cnt = jnp.sum((ties & (col <= mid)).astype(jnp.int32),
                      axis=1, keepdims=True)
        ok = cnt >= need
        return jnp.where(ok, lo2, mid + 1), jnp.where(ok, mid, hi2)

    c_sel, _ = jax.lax.fori_loop(
        0, col_iters, body2,
        (jnp.zeros_like(cnt_gt), jnp.full_like(cnt_gt, width - 1)))
    sel = gt | (ties & (col <= c_sel))
    o_ref[...] = sel.astype(jnp.int8)


def _p3_sdpa(q_ref, k_ref, v_ref, mask_ref, o_ref):
    keep = mask_ref[...] != 0
    for h in range(_HG):
        sl = slice(h * D_K, (h + 1) * D_K)
        s = jax.lax.dot_general(q_ref[:, sl], k_ref[:, sl],
                                (((1,), (1,)), ((), ())),
                                preferred_element_type=jnp.float32)
        s = jnp.where(keep, s / jnp.sqrt(jnp.float32(D_K)), jnp.float32(-1e30))
        mx = jnp.max(s, axis=1, keepdims=True)
        p = jnp.exp(s - mx)
        l = jnp.sum(p, axis=1, keepdims=True)
        o_ref[:, sl] = jnp.dot(p, v_ref[:, sl],
                               preferred_element_type=jnp.float32) / l


def _p3_sdpa_dense(q_ref, k_ref, v_ref, o_ref):
    for h in range(_HG):
        sl = slice(h * D_K, (h + 1) * D_K)
        s = jax.lax.dot_general(q_ref[:, sl], k_ref[:, sl],
                                (((1,), (1,)), ((), ())),
                                preferred_element_type=jnp.float32)
        s = s / jnp.sqrt(jnp.float32(D_K))
        mx = jnp.max(s, axis=1, keepdims=True)
        p = jnp.exp(s - mx)
        l = jnp.sum(p, axis=1, keepdims=True)
        o_ref[:, sl] = jnp.dot(p, v_ref[:, sl],
                               preferred_element_type=jnp.float32) / l


def _p4_matmul(a1_ref, a2_ref, a3_ref, w_ref, o_ref):
    w = w_ref[...]
    o_ref[:512, :] = jnp.dot(a1_ref[...], w,
                             preferred_element_type=jnp.float32)
    o_ref[512:1024, :] = jnp.dot(a2_ref[...], w,
                                 preferred_element_type=jnp.float32)
    o_ref[1024:, :] = jnp.dot(a3_ref[...], w,
                              preferred_element_type=jnp.float32)


def kernel(x, Wqkv, Wo, Wq_idx, Wk_idx, Ww_idx):
    b, s, _ = x.shape
    x2 = x[0]

    WT = jnp.concatenate([Wqkv, Wq_idx, Wk_idx, Ww_idx], axis=0)
    WT = jnp.pad(WT, ((0, _N_PAD - _N_COLS), (0, 0))).T

    theta = 1.0 / (10000.0 ** (jnp.arange(0, D_K, 2, dtype=jnp.float32) / D_K))
    pos = jnp.arange(SEQ, dtype=jnp.float32)
    freqs = pos[:, None] * theta[None, :]
    cos2 = jnp.repeat(jnp.cos(freqs), 2, axis=1)
    sin2 = jnp.repeat(jnp.sin(freqs), 2, axis=1)
    cos512 = jnp.tile(cos2, (1, _BN // D_K))
    sin512 = jnp.tile(sin2, (1, _BN // D_K))
    r128 = (jnp.zeros((D_K, D_K), jnp.float32)
            .at[2 * jnp.arange(64) + 1, 2 * jnp.arange(64)].set(-1.0)
            .at[2 * jnp.arange(64), 2 * jnp.arange(64) + 1].set(1.0))
    r512 = jax.scipy.linalg.block_diag(*([r128] * (_BN // D_K)))

    proj = pl.pallas_call(
        _p1_proj_rope,
        grid=(_N_PAD // _BN,),
        in_specs=[
            pl.BlockSpec((SEQ, D_MODEL), lambda n: (0, 0)),
            pl.BlockSpec((D_MODEL, _BN), lambda n: (0, n)),
            pl.BlockSpec((SEQ, _BN), lambda n: (0, 0)),
            pl.BlockSpec((SEQ, _BN), lambda n: (0, 0)),
            pl.BlockSpec((_BN, _BN), lambda n: (0, 0)),
        ],
        out_specs=pl.BlockSpec((SEQ, _BN), lambda n: (0, n)),
        out_shape=jax.ShapeDtypeStruct((SEQ, _N_PAD), jnp.float32),
    )(x2, WT, cos512, sin512, r512)

    mask_b = pl.pallas_call(
        functools.partial(_p2_index_topk, row0=512, width=1024, col_iters=10),
        grid=(2,),
        in_specs=[
            pl.BlockSpec((_BM, _BN), lambda m: (2 + m, 12)),
            pl.BlockSpec((1024, D_K), lambda m: (0, 50)),
        ],
        out_specs=pl.BlockSpec((_BM, 1024), lambda m: (m, 0)),
        out_shape=jax.ShapeDtypeStruct((512, 1024), jnp.int8),
    )(proj, proj)

    mask_c = pl.pallas_call(
        functools.partial(_p2_index_topk, row0=1024, width=2048, col_iters=11),
        grid=(4,),
        in_specs=[
            pl.BlockSpec((_BM, _BN), lambda m: (4 + m, 12)),
            pl.BlockSpec((SEQ, D_K), lambda m: (0, 50)),
        ],
        out_specs=pl.BlockSpec((_BM, 2048), lambda m: (m, 0)),
        out_shape=jax.ShapeDtypeStruct((1024, 2048), jnp.int8),
    )(proj, proj)

    attn_a = pl.pallas_call(
        _p3_sdpa_dense,
        grid=(N_HEADS // _HG, 2),
        in_specs=[
            pl.BlockSpec((_BM, _HGW), lambda g, m: (m, g)),
            pl.BlockSpec((512, _HGW), lambda g, m: (0, 2 + g)),
            pl.BlockSpec((512, _HGW), lambda g, m: (0, 4 + g)),
        ],
        out_specs=pl.BlockSpec((_BM, _HGW), lambda g, m: (m, g)),
        out_shape=jax.ShapeDtypeStruct((512, D_MODEL), jnp.float32),
    )(proj, proj, proj)

    attn_b = pl.pallas_call(
        _p3_sdpa,
        grid=(N_HEADS // _HG, 2),
        in_specs=[
            pl.BlockSpec((_BM, _HGW), lambda g, m: (2 + m, g)),
            pl.BlockSpec((1024, _HGW), lambda g, m: (0, 2 + g)),
            pl.BlockSpec((1024, _HGW), lambda g, m: (0, 4 + g)),
            pl.BlockSpec((_BM, 1024), lambda g, m: (m, 0)),
        ],
        out_specs=pl.BlockSpec((_BM, _HGW), lambda g, m: (m, g)),
        out_shape=jax.ShapeDtypeStruct((512, D_MODEL), jnp.float32),
    )(proj, proj, proj, mask_b)

    attn_c = pl.pallas_call(
        _p3_sdpa,
        grid=(N_HEADS // _HG, 4),
        in_specs=[
            pl.BlockSpec((_BM, _HGW), lambda g, m: (4 + m, g)),
            pl.BlockSpec((SEQ, _HGW), lambda g, m: (0, 2 + g)),
            pl.BlockSpec((SEQ, _HGW), lambda g, m: (0, 4 + g)),
            pl.BlockSpec((_BM, SEQ), lambda g, m: (m, 0)),
        ],
        out_specs=pl.BlockSpec((_BM, _HGW), lambda g, m: (m, g)),
        out_shape=jax.ShapeDtypeStruct((1024, D_MODEL), jnp.float32),
    )(proj, proj, proj, mask_c)

    out = pl.pallas_call(
        _p4_matmul,
        grid=(D_MODEL // _BN,),
        in_specs=[
            pl.BlockSpec((512, D_MODEL), lambda n: (0, 0)),
            pl.BlockSpec((512, D_MODEL), lambda n: (0, 0)),
            pl.BlockSpec((1024, D_MODEL), lambda n: (0, 0)),
            pl.BlockSpec((D_MODEL, _BN), lambda n: (0, n)),
        ],
        out_specs=pl.BlockSpec((SEQ, _BN), lambda n: (0, n)),
        out_shape=jax.ShapeDtypeStruct((SEQ, D_MODEL), jnp.float32),
    )(attn_a, attn_b, attn_c, Wo.T)

    return out.reshape(b, s, D_MODEL)


# speedup vs baseline: 18.7427x; 1.4405x over previous
import functools

import jax
import jax.numpy as jnp
import numpy as np
from jax.experimental import pallas as pl

D_MODEL = 2048
N_HEADS = 16
D_K = 128
SEQ = 2048
HI = 4
DI = 64
K_SEL = 512

_QI = HI * DI
_N_COLS = 3 * D_MODEL + _QI + DI + HI
_N_PAD = 6656
_BM = 256
_BN = 512
_HG = 8
_HGW = _HG * D_K

_theta = 1.0 / (10000.0 ** (np.arange(0, D_K, 2, dtype=np.float32) / D_K))
_freqs = np.arange(SEQ, dtype=np.float32)[:, None] * _theta[None, :]
_COS512 = np.tile(np.repeat(np.cos(_freqs).astype(np.float32), 2, axis=1),
                  (1, _BN // D_K))
_SIN512 = np.tile(np.repeat(np.sin(_freqs).astype(np.float32), 2, axis=1),
                  (1, _BN // D_K))
_R128 = np.kron(np.eye(64, dtype=np.float32),
                np.array([[0.0, 1.0], [-1.0, 0.0]], dtype=np.float32))
_R512 = np.kron(np.eye(_BN // D_K, dtype=np.float32), _R128)


def _mm(a, b):
    return jax.lax.dot_general(a, b, (((1,), (1,)), ((), ())),
                               preferred_element_type=jnp.float32)


def _p1_proj_rope(x_ref, w_ref, widx_ref, cos_ref, sin_ref, rot_ref, o_ref):
    n = pl.program_id(0)

    @pl.when(n < 8)
    def _():
        acc = _mm(x_ref[...], w_ref[...])
        rot = jnp.dot(acc, rot_ref[...], preferred_element_type=jnp.float32)
        o_ref[...] = acc * cos_ref[...] + rot * sin_ref[...]

    @pl.when((n >= 8) & (n < 12))
    def _():
        o_ref[...] = _mm(x_ref[...], w_ref[...])

    @pl.when(n == 12)
    def _():
        o_ref[...] = _mm(x_ref[...], widx_ref[...])


def _p2_index_topk(qw_ref, ki_ref, o_ref, *, row0, width, col_iters):
    qw = qw_ref[...]
    ki = ki_ref[:, :DI]
    acc = None
    for h in range(HI):
        qh = qw[:, h * DI:(h + 1) * DI]
        d = jax.lax.dot_general(qh, ki, (((1,), (1,)), ((), ())),
                                preferred_element_type=jnp.float32)
        a = jnp.maximum(d, 0.0) * qw[:, _QI + DI + h:_QI + DI + h + 1]
        acc = a if acc is None else acc + a

    m = pl.program_id(0)
    row = row0 + m * _BM + jax.lax.broadcasted_iota(jnp.int32, (_BM, width), 0)
    col = jax.lax.broadcasted_iota(jnp.int32, (_BM, width), 1)
    scores = jnp.where(col > row, jnp.float32(-1e9), acc)

    z = jnp.where(scores == 0.0, jnp.float32(0.0), scores)
    bits = jax.lax.bitcast_convert_type(z, jnp.int32)
    keys = jnp.where(bits < 0, bits ^ jnp.int32(0x7FFFFFFF), bits)

    lo = jnp.min(keys, axis=1, keepdims=True)
    hi = jnp.max(keys, axis=1, keepdims=True)

    def body(_, lh):
        lo, hi = lh
        x = lo ^ hi
        mid = (lo & hi) + (x >> 1) + (x & 1)
        cnt = jnp.sum((keys >= mid).astype(jnp.int32), axis=1, keepdims=True)
        take = cnt >= K_SEL
        return jnp.where(take, mid, lo), jnp.where(take, hi, mid - 1)

    tau, _ = jax.lax.fori_loop(0, 32, body, (lo, hi))

    gt = keys > tau
    cnt_gt = jnp.sum(gt.astype(jnp.int32), axis=1, keepdims=True)
    ties = keys == tau
    need = K_SEL - cnt_gt

    def body2(_, lh):
        lo2, hi2 = lh
        mid = (lo2 + hi2) >> 1
        cnt = jnp.sum((ties & (col <= mid)).astype(jnp.int32),
                      axis=1, keepdims=True)
        ok = cnt >= need
        return jnp.where(ok, lo2, mid + 1), jnp.where(ok, mid, hi2)

    c_sel, _ = jax.lax.fori_loop(
        0, col_iters, body2,
        (jnp.zeros_like(cnt_gt), jnp.full_like(cnt_gt, width - 1)))
    sel = gt | (ties & (col <= c_sel))
    o_ref[...] = sel.astype(jnp.int8)


def _p3_sdpa(q_ref, k_ref, v_ref, mask_ref, o_ref):
    keep = mask_ref[...] != 0
    for h in range(_HG):
        sl = slice(h * D_K, (h + 1) * D_K)
        s = jax.lax.dot_general(q_ref[:, sl], k_ref[:, sl],
                                (((1,), (1,)), ((), ())),
                                preferred_element_type=jnp.float32)
        s = jnp.where(keep, s / jnp.sqrt(jnp.float32(D_K)), jnp.float32(-1e30))
        mx = jnp.max(s, axis=1, keepdims=True)
        p = jnp.exp(s - mx)
        l = jnp.sum(p, axis=1, keepdims=True)
        o_ref[:, sl] = jnp.dot(p, v_ref[:, sl],
                               preferred_element_type=jnp.float32) / l


def _p3_sdpa_dense(q_ref, k_ref, v_ref, o_ref):
    for h in range(_HG):
        sl = slice(h * D_K, (h + 1) * D_K)
        s = jax.lax.dot_general(q_ref[:, sl], k_ref[:, sl],
                                (((1,), (1,)), ((), ())),
                                preferred_element_type=jnp.float32)
        s = s / jnp.sqrt(jnp.float32(D_K))
        mx = jnp.max(s, axis=1, keepdims=True)
        p = jnp.exp(s - mx)
        l = jnp.sum(p, axis=1, keepdims=True)
        o_ref[:, sl] = jnp.dot(p, v_ref[:, sl],
                               preferred_element_type=jnp.float32) / l


def _p4_matmul(a1_ref, a2_ref, a3_ref, w_ref, o_ref):
    w = w_ref[...]
    o_ref[:512, :] = _mm(a1_ref[...], w)
    o_ref[512:1024, :] = _mm(a2_ref[...], w)
    o_ref[1024:, :] = _mm(a3_ref[...], w)


def kernel(x, Wqkv, Wo, Wq_idx, Wk_idx, Ww_idx):
    b, s, _ = x.shape
    x2 = x[0]

    widx = jnp.pad(jnp.concatenate([Wq_idx, Wk_idx, Ww_idx], axis=0),
                   ((0, _BN - (_QI + DI + HI)), (0, 0)))

    proj = pl.pallas_call(
        _p1_proj_rope,
        grid=(_N_PAD // _BN,),
        in_specs=[
            pl.BlockSpec((SEQ, D_MODEL), lambda n: (0, 0)),
            pl.BlockSpec((_BN, D_MODEL), lambda n: (jnp.minimum(n, 11), 0)),
            pl.BlockSpec((_BN, D_MODEL), lambda n: (0, 0)),
            pl.BlockSpec((SEQ, _BN), lambda n: (0, 0)),
            pl.BlockSpec((SEQ, _BN), lambda n: (0, 0)),
            pl.BlockSpec((_BN, _BN), lambda n: (0, 0)),
        ],
        out_specs=pl.BlockSpec((SEQ, _BN), lambda n: (0, n)),
        out_shape=jax.ShapeDtypeStruct((SEQ, _N_PAD), jnp.float32),
    )(x2, Wqkv, widx, jnp.asarray(_COS512), jnp.asarray(_SIN512),
      jnp.asarray(_R512))

    mask_b = pl.pallas_call(
        functools.partial(_p2_index_topk, row0=512, width=1024, col_iters=10),
        grid=(2,),
        in_specs=[
            pl.BlockSpec((_BM, _BN), lambda m: (2 + m, 12)),
            pl.BlockSpec((1024, D_K), lambda m: (0, 50)),
        ],
        out_specs=pl.BlockSpec((_BM, 1024), lambda m: (m, 0)),
        out_shape=jax.ShapeDtypeStruct((512, 1024), jnp.int8),
    )(proj, proj)

    mask_c = pl.pallas_call(
        functools.partial(_p2_index_topk, row0=1024, width=2048, col_iters=11),
        grid=(4,),
        in_specs=[
            pl.BlockSpec((_BM, _BN), lambda m: (4 + m, 12)),
            pl.BlockSpec((SEQ, D_K), lambda m: (0, 50)),
        ],
        out_specs=pl.BlockSpec((_BM, 2048), lambda m: (m, 0)),
        out_shape=jax.ShapeDtypeStruct((1024, 2048), jnp.int8),
    )(proj, proj)

    attn_a = pl.pallas_call(
        _p3_sdpa_dense,
        grid=(N_HEADS // _HG, 2),
        in_specs=[
            pl.BlockSpec((_BM, _HGW), lambda g, m: (m, g)),
            pl.BlockSpec((512, _HGW), lambda g, m: (0, 2 + g)),
            pl.BlockSpec((512, _HGW), lambda g, m: (0, 4 + g)),
        ],
        out_specs=pl.BlockSpec((_BM, _HGW), lambda g, m: (m, g)),
        out_shape=jax.ShapeDtypeStruct((512, D_MODEL), jnp.float32),
    )(proj, proj, proj)

    attn_b = pl.pallas_call(
        _p3_sdpa,
        grid=(N_HEADS // _HG, 2),
        in_specs=[
            pl.BlockSpec((_BM, _HGW), lambda g, m: (2 + m, g)),
            pl.BlockSpec((1024, _HGW), lambda g, m: (0, 2 + g)),
            pl.BlockSpec((1024, _HGW), lambda g, m: (0, 4 + g)),
            pl.BlockSpec((_BM, 1024), lambda g, m: (m, 0)),
        ],
        out_specs=pl.BlockSpec((_BM, _HGW), lambda g, m: (m, g)),
        out_shape=jax.ShapeDtypeStruct((512, D_MODEL), jnp.float32),
    )(proj, proj, proj, mask_b)

    attn_c = pl.pallas_call(
        _p3_sdpa,
        grid=(N_HEADS // _HG, 4),
        in_specs=[
            pl.BlockSpec((_BM, _HGW), lambda g, m: (4 + m, g)),
            pl.BlockSpec((SEQ, _HGW), lambda g, m: (0, 2 + g)),
            pl.BlockSpec((SEQ, _HGW), lambda g, m: (0, 4 + g)),
            pl.BlockSpec((_BM, SEQ), lambda g, m: (m, 0)),
        ],
        out_specs=pl.BlockSpec((_BM, _HGW), lambda g, m: (m, g)),
        out_shape=jax.ShapeDtypeStruct((1024, D_MODEL), jnp.float32),
    )(proj, proj, proj, mask_c)

    out = pl.pallas_call(
        _p4_matmul,
        grid=(D_MODEL // _BN,),
        in_specs=[
            pl.BlockSpec((512, D_MODEL), lambda n: (0, 0)),
            pl.BlockSpec((512, D_MODEL), lambda n: (0, 0)),
            pl.BlockSpec((1024, D_MODEL), lambda n: (0, 0)),
            pl.BlockSpec((_BN, D_MODEL), lambda n: (n, 0)),
        ],
        out_specs=pl.BlockSpec((SEQ, _BN), lambda n: (0, n)),
        out_shape=jax.ShapeDtypeStruct((SEQ, D_MODEL), jnp.float32),
    )(attn_a, attn_b, attn_c, Wo)

    return out.reshape(b, s, D_MODEL)
